# baseline (speedup 1.0000x reference)
import functools

import jax
import jax.numpy as jnp
from jax import lax
from jax.experimental import pallas as pl
from jax.experimental.pallas import tpu as pltpu
from jax.experimental.pallas import tpu_sc as plsc

NUM_PARTS = 256
NUM_K = 2047
STRIDE = NUM_K + 1
VAE_DIMS = 32
OUT_DIMS = 32
BATCH = 1024

TABLE_ROWS = STRIDE * NUM_PARTS
NW = 32
ROWS_TOTAL = BATCH * NUM_PARTS
ROWS_PER_W = ROWS_TOTAL // NW
IDX_ROWS = ROWS_PER_W // 128
CHUNK = 1024
NCHUNK = ROWS_PER_W // CHUNK
GPC = CHUNK // 128


A_COLS = 4096


def _project_body(x_ref, w_ref, peb_ref, o_ref):
    i = pl.program_id(0)
    x = x_ref[...]
    acc = lax.dot_general(x, w_ref[...], (((0,), (0,)), ((), ())),
                          preferred_element_type=jnp.float32)
    peb2 = peb_ref[pl.ds(2 * i, 2), :]
    acc = acc + jnp.repeat(peb2, STRIDE, axis=0)
    acc3 = acc.reshape(A_COLS // 4, 4, VAE_DIMS)
    o_ref[...] = jnp.concatenate([acc3[:, a, :] for a in range(4)], axis=1)


def _project(emb_t, w, peb):
    return pl.pallas_call(
        _project_body,
        grid=(TABLE_ROWS // A_COLS,),
        in_specs=[
            pl.BlockSpec((VAE_DIMS, A_COLS), lambda i: (0, i)),
            pl.BlockSpec((VAE_DIMS, OUT_DIMS), lambda i: (0, 0)),
            pl.BlockSpec((NUM_PARTS, OUT_DIMS), lambda i: (0, 0)),
        ],
        out_specs=pl.BlockSpec((A_COLS // 4, 128), lambda i: (i, 0)),
        out_shape=jax.ShapeDtypeStruct((TABLE_ROWS // 4, 128), jnp.float32),
    )(emb_t, w, peb)


def _sc_gather_body(h2d, table, mu_out, idx2d, rows0, rows1, gsem, wsem):
    cid = lax.axis_index("c")
    sid = lax.axis_index("s")
    wid = sid * 2 + cid
    pltpu.sync_copy(h2d.at[pl.ds(wid * IDX_ROWS, IDX_ROWS)], idx2d)

    lane = lax.iota(jnp.int32, 16)

    def add_offs(j, carry):
        row = j // 8
        col = (j % 8) * 16
        offs = ((j % 16) * 16 + lane) * STRIDE
        v = idx2d[row, pl.ds(col, 16)]
        idx2d[row, pl.ds(col, 16)] = v + offs
        return carry

    lax.fori_loop(0, IDX_ROWS * 8, add_offs, 0)

    rows = [rows0, rows1]
    base = wid * ROWS_PER_W
    wb_handles = [None, None]
    for c in range(NCHUNK):
        buf = rows[c % 2]
        if wb_handles[c % 2] is not None:
            wb_handles[c % 2].wait()
        ghandles = []
        for k in range(GPC):
            ghandles.append(pltpu.async_copy(
                table.at[idx2d.at[c * GPC + k]],
                buf.at[pl.ds(k * 128, 128)],
                gsem))
        for h in ghandles:
            h.wait()
        wb_handles[c % 2] = pltpu.async_copy(
            buf, mu_out.at[pl.ds(base + c * CHUNK, CHUNK)], wsem)
    for h in wb_handles:
        if h is not None:
            h.wait()


def _sc_gather(h2d, table):
    mesh = plsc.VectorSubcoreMesh(core_axis_name="c", subcore_axis_name="s")
    f = functools.partial(
        pl.kernel,
        mesh=mesh,
        compiler_params=pltpu.CompilerParams(use_tc_tiling_on_sc=False),
        out_type=jax.ShapeDtypeStruct((ROWS_TOTAL, VAE_DIMS), jnp.float32),
        scratch_types=[
            pltpu.VMEM((IDX_ROWS, 128), jnp.int32),
            pltpu.VMEM((CHUNK, VAE_DIMS), jnp.float32),
            pltpu.VMEM((CHUNK, VAE_DIMS), jnp.float32),
            pltpu.SemaphoreType.DMA,
            pltpu.SemaphoreType.DMA,
        ],
    )(_sc_gather_body)
    return f(h2d, table)


C_BLK = 1024


def _finish_body(m_ref, o_ref):
    m = m_ref[...]
    m4 = m.reshape(16, 64, 4, VAE_DIMS)
    o_ref[...] = jnp.transpose(m4, (0, 3, 1, 2)).reshape(16, OUT_DIMS, NUM_PARTS)


def _finish(mu4):
    return pl.pallas_call(
        _finish_body,
        grid=(ROWS_TOTAL // 4 // C_BLK,),
        in_specs=[pl.BlockSpec((C_BLK, 128), lambda i: (i, 0))],
        out_specs=pl.BlockSpec((16, OUT_DIMS, NUM_PARTS), lambda i: (i, 0, 0)),
        out_shape=jax.ShapeDtypeStruct((BATCH, OUT_DIMS, NUM_PARTS), jnp.float32),
    )(mu4)


def kernel(hashes, embedding, pe, W, b):
    B, P = hashes.shape
    h2d = hashes.reshape(-1, 128)
    emb_t = embedding.T
    peb = pe + b[None, :]
    p4 = _project(emb_t, W, peb)
    table = p4.reshape(TABLE_ROWS, VAE_DIMS)
    mu = _sc_gather(h2d, table)
    mu4 = mu.reshape(ROWS_TOTAL // 4, 128)
    out_t = _finish(mu4)
    return jnp.swapaxes(out_t, 1, 2)


# speedup vs baseline: 3.0471x; 3.0471x over previous
import functools

import jax
import jax.numpy as jnp
from jax import lax
from jax.experimental import pallas as pl
from jax.experimental.pallas import tpu as pltpu
from jax.experimental.pallas import tpu_sc as plsc

NUM_PARTS = 256
NUM_K = 2047
STRIDE = NUM_K + 1
VAE_DIMS = 32
OUT_DIMS = 32
BATCH = 1024

TABLE_ROWS = STRIDE * NUM_PARTS
NW = 32
ROWS_TOTAL = BATCH * NUM_PARTS
ROWS_PER_W = ROWS_TOTAL // NW
IDX_ROWS = ROWS_PER_W // 128
CHUNK = 1024
NCHUNK = ROWS_PER_W // CHUNK
GPC = CHUNK // 128


A_COLS = 4096


def _project_body(x_ref, wt_ref, peb_ref, o_ref):
    i = pl.program_id(0)
    x = x_ref[...]
    acc_t = jnp.dot(wt_ref[...], x, preferred_element_type=jnp.float32)
    prow = peb_ref[pl.ds(2 * i, 2), :]
    for a in range(4):
        pa = jnp.swapaxes(acc_t[:, 1024 * a:1024 * (a + 1)], 0, 1)
        o_ref[:, 32 * a:32 * (a + 1)] = pa + prow[(a // 2):(a // 2) + 1, :]


def _project(emb_t, wt, peb):
    return pl.pallas_call(
        _project_body,
        grid=(TABLE_ROWS // A_COLS,),
        in_specs=[
            pl.BlockSpec((VAE_DIMS, A_COLS), lambda i: (0, i)),
            pl.BlockSpec((OUT_DIMS, VAE_DIMS), lambda i: (0, 0)),
            pl.BlockSpec((NUM_PARTS, OUT_DIMS), lambda i: (0, 0)),
        ],
        out_specs=pl.BlockSpec((A_COLS // 4, 128), lambda i: (i, 0)),
        out_shape=jax.ShapeDtypeStruct((TABLE_ROWS // 4, 128), jnp.float32),
    )(emb_t, wt, peb)


def _sc_gather_body(h2d, table, mu_out, idx2d, rows0, rows1, gsem, wsem):
    cid = lax.axis_index("c")
    sid = lax.axis_index("s")
    wid = sid * 2 + cid
    pltpu.sync_copy(h2d.at[pl.ds(wid * IDX_ROWS, IDX_ROWS)], idx2d)

    lane = lax.iota(jnp.int32, 16)

    def add_offs(j, carry):
        row = j // 8
        col = (j % 8) * 16
        offs = ((j % 16) * 16 + lane) * STRIDE
        r = idx2d[row, pl.ds(col, 16)] + offs
        w = jnp.bitwise_and(r, 4095)
        rp = (r - w) + jnp.left_shift(jnp.bitwise_and(w, 1023), 2) \
            + jnp.right_shift(w, 10)
        idx2d[row, pl.ds(col, 16)] = rp
        return carry

    lax.fori_loop(0, IDX_ROWS * 8, add_offs, 0)

    rows = [rows0, rows1]
    wb_handles = [None, None]
    for c in range(NCHUNK):
        buf = rows[c % 2]
        if wb_handles[c % 2] is not None:
            wb_handles[c % 2].wait()
        ghandles = []
        for k in range(GPC):
            ghandles.append(pltpu.async_copy(
                table.at[idx2d.at[c * GPC + k]],
                buf.at[pl.ds(k * 128, 128)],
                gsem))
        for h in ghandles:
            h.wait()
        j = 2 * wid + (c // 4)
        a = c % 4
        wb_handles[c % 2] = pltpu.async_copy(
            buf,
            mu_out.at[pl.ds(j * 1024, CHUNK), pl.ds(32 * a, VAE_DIMS)],
            wsem)
    for h in wb_handles:
        if h is not None:
            h.wait()


def _sc_gather(h2d, table):
    mesh = plsc.VectorSubcoreMesh(core_axis_name="c", subcore_axis_name="s")
    f = functools.partial(
        pl.kernel,
        mesh=mesh,
        compiler_params=pltpu.CompilerParams(use_tc_tiling_on_sc=False),
        out_type=jax.ShapeDtypeStruct((ROWS_TOTAL // 4, 128), jnp.float32),
        scratch_types=[
            pltpu.VMEM((IDX_ROWS, 128), jnp.int32),
            pltpu.VMEM((CHUNK, VAE_DIMS), jnp.float32),
            pltpu.VMEM((CHUNK, VAE_DIMS), jnp.float32),
            pltpu.SemaphoreType.DMA,
            pltpu.SemaphoreType.DMA,
        ],
    )(_sc_gather_body)
    return f(h2d, table)


def _finish_body(m_ref, o_ref):
    m = m_ref[...]
    for a in range(4):
        qa = m[:, 32 * a:32 * (a + 1)]
        o_ref[4 * a:4 * (a + 1)] = jnp.swapaxes(
            qa.reshape(4, NUM_PARTS, VAE_DIMS), 1, 2)


def _finish(mu4):
    return pl.pallas_call(
        _finish_body,
        grid=(ROWS_TOTAL // A_COLS,),
        in_specs=[pl.BlockSpec((1024, 128), lambda i: (i, 0))],
        out_specs=pl.BlockSpec((16, OUT_DIMS, NUM_PARTS), lambda i: (i, 0, 0)),
        out_shape=jax.ShapeDtypeStruct((BATCH, OUT_DIMS, NUM_PARTS), jnp.float32),
    )(mu4)


def kernel(hashes, embedding, pe, W, b):
    B, P = hashes.shape
    h2d = hashes.reshape(-1, 128)
    emb_t = embedding.T
    peb = pe + b[None, :]
    p4 = _project(emb_t, W.T, peb)
    table = p4.reshape(TABLE_ROWS, VAE_DIMS)
    mu4 = _sc_gather(h2d, table)
    out_t = _finish(mu4)
    return jnp.swapaxes(out_t, 1, 2)


# speedup vs baseline: 4.0858x; 1.3409x over previous
import functools

import jax
import jax.numpy as jnp
from jax import lax
from jax.experimental import pallas as pl
from jax.experimental.pallas import tpu as pltpu
from jax.experimental.pallas import tpu_sc as plsc

NUM_PARTS = 256
NUM_K = 2047
STRIDE = NUM_K + 1
VAE_DIMS = 32
OUT_DIMS = 32
BATCH = 1024

TABLE_ROWS = STRIDE * NUM_PARTS
NW = 32
ROWS_TOTAL = BATCH * NUM_PARTS
ROWS_PER_W = ROWS_TOTAL // NW
IDX_ROWS = ROWS_PER_W // 128
CHUNK = 1024
NCHUNK = ROWS_PER_W // CHUNK
GPC = CHUNK // 128


A_COLS = 4096


def _project_body(x_ref, wt_ref, peb_ref, o_ref):
    i = pl.program_id(0)
    x = x_ref[...]
    acc_t = jnp.dot(wt_ref[...], x, preferred_element_type=jnp.float32)
    prow = peb_ref[pl.ds(2 * i, 2), :]
    stacked = jnp.concatenate(
        [acc_t[:, 1024 * a:1024 * (a + 1)] for a in range(4)], axis=0)
    pebrow = jnp.concatenate(
        [prow[0:1], prow[0:1], prow[1:2], prow[1:2]], axis=1)
    o_ref[...] = jnp.swapaxes(stacked, 0, 1) + pebrow


def _project(emb_t, wt, peb):
    return pl.pallas_call(
        _project_body,
        grid=(TABLE_ROWS // A_COLS,),
        in_specs=[
            pl.BlockSpec((VAE_DIMS, A_COLS), lambda i: (0, i)),
            pl.BlockSpec((OUT_DIMS, VAE_DIMS), lambda i: (0, 0)),
            pl.BlockSpec((NUM_PARTS, OUT_DIMS), lambda i: (0, 0)),
        ],
        out_specs=pl.BlockSpec((A_COLS // 4, 128), lambda i: (i, 0)),
        out_shape=jax.ShapeDtypeStruct((TABLE_ROWS // 4, 128), jnp.float32),
    )(emb_t, wt, peb)


def _sc_gather_body(h2d, table, mu_out, idx2d, rows0, rows1, gsem, wsem):
    cid = lax.axis_index("c")
    sid = lax.axis_index("s")
    wid = sid * 2 + cid
    pltpu.sync_copy(h2d.at[pl.ds(wid * IDX_ROWS, IDX_ROWS)], idx2d)

    lane = lax.iota(jnp.int32, 16)

    def add_offs(j, carry):
        row = j // 8
        col = (j % 8) * 16
        offs = ((j % 16) * 16 + lane) * STRIDE
        r = idx2d[row, pl.ds(col, 16)] + offs
        w = jnp.bitwise_and(r, 4095)
        rp = (r - w) + jnp.left_shift(jnp.bitwise_and(w, 1023), 2) \
            + jnp.right_shift(w, 10)
        idx2d[row, pl.ds(col, 16)] = rp
        return carry

    lax.fori_loop(0, IDX_ROWS * 8, add_offs, 0)

    rows = [rows0, rows1]
    wb_handles = [None, None]
    for c in range(NCHUNK):
        buf = rows[c % 2]
        if wb_handles[c % 2] is not None:
            wb_handles[c % 2].wait()
        ghandles = []
        for k in range(GPC):
            ghandles.append(pltpu.async_copy(
                table.at[idx2d.at[c * GPC + k]],
                buf.at[pl.ds(k * 128, 128)],
                gsem))
        for h in ghandles:
            h.wait()
        j = 2 * wid + (c // 4)
        a = c % 4
        wb_handles[c % 2] = pltpu.async_copy(
            buf,
            mu_out.at[pl.ds(j * 1024, CHUNK), pl.ds(32 * a, VAE_DIMS)],
            wsem)
    for h in wb_handles:
        if h is not None:
            h.wait()


def _sc_gather(h2d, table):
    mesh = plsc.VectorSubcoreMesh(core_axis_name="c", subcore_axis_name="s")
    f = functools.partial(
        pl.kernel,
        mesh=mesh,
        compiler_params=pltpu.CompilerParams(use_tc_tiling_on_sc=False),
        out_type=jax.ShapeDtypeStruct((ROWS_TOTAL // 4, 128), jnp.float32),
        scratch_types=[
            pltpu.VMEM((IDX_ROWS, 128), jnp.int32),
            pltpu.VMEM((CHUNK, VAE_DIMS), jnp.float32),
            pltpu.VMEM((CHUNK, VAE_DIMS), jnp.float32),
            pltpu.SemaphoreType.DMA,
            pltpu.SemaphoreType.DMA,
        ],
    )(_sc_gather_body)
    return f(h2d, table)


def _finish_body(m_ref, o_ref):
    m = m_ref[...]
    t = jnp.swapaxes(m, 0, 1)
    for a in range(4):
        for bb in range(4):
            o_ref[4 * a + bb] = t[32 * a:32 * (a + 1),
                                  256 * bb:256 * (bb + 1)]


def _finish(mu4):
    return pl.pallas_call(
        _finish_body,
        grid=(ROWS_TOTAL // A_COLS,),
        in_specs=[pl.BlockSpec((1024, 128), lambda i: (i, 0))],
        out_specs=pl.BlockSpec((16, OUT_DIMS, NUM_PARTS), lambda i: (i, 0, 0)),
        out_shape=jax.ShapeDtypeStruct((BATCH, OUT_DIMS, NUM_PARTS), jnp.float32),
    )(mu4)


def kernel(hashes, embedding, pe, W, b):
    B, P = hashes.shape
    h2d = hashes.reshape(-1, 128)
    emb_t = embedding.T
    peb = pe + b[None, :]
    p4 = _project(emb_t, W.T, peb)
    table = p4.reshape(TABLE_ROWS, VAE_DIMS)
    mu4 = _sc_gather(h2d, table)
    out_t = _finish(mu4)
    return jnp.swapaxes(out_t, 1, 2)


# speedup vs baseline: 5.2503x; 1.2850x over previous
import functools

import jax
import jax.numpy as jnp
from jax import lax
from jax.experimental import pallas as pl
from jax.experimental.pallas import tpu as pltpu
from jax.experimental.pallas import tpu_sc as plsc

NUM_PARTS = 256
NUM_K = 2047
STRIDE = NUM_K + 1
VAE_DIMS = 32
OUT_DIMS = 32
BATCH = 1024

TABLE_ROWS = STRIDE * NUM_PARTS
NW = 32
ROWS_TOTAL = BATCH * NUM_PARTS
ROWS_PER_W = ROWS_TOTAL // NW
IDX_ROWS = ROWS_PER_W // 128
CHUNK = 1024
NCHUNK = ROWS_PER_W // CHUNK
GPC = CHUNK // 128


A_COLS = 8192
AQ = A_COLS // 4


def _project_body(x_ref, wt_ref, peb_ref, o_ref):
    i = pl.program_id(0)
    x = x_ref[...]
    acc_t = jnp.dot(wt_ref[...], x, preferred_element_type=jnp.float32)
    prow = peb_ref[pl.ds(4 * i, 4), :]
    stacked = jnp.concatenate(
        [acc_t[:, AQ * a:AQ * (a + 1)] for a in range(4)], axis=0)
    pebrow = jnp.concatenate(
        [prow[a:a + 1] for a in range(4)], axis=1)
    o_ref[...] = jnp.swapaxes(stacked, 0, 1) + pebrow


def _project(emb_t, wt, peb):
    return pl.pallas_call(
        _project_body,
        grid=(TABLE_ROWS // A_COLS,),
        in_specs=[
            pl.BlockSpec((VAE_DIMS, A_COLS), lambda i: (0, i)),
            pl.BlockSpec((OUT_DIMS, VAE_DIMS), lambda i: (0, 0)),
            pl.BlockSpec((NUM_PARTS, OUT_DIMS), lambda i: (0, 0)),
        ],
        out_specs=pl.BlockSpec((AQ, 128), lambda i: (i, 0)),
        out_shape=jax.ShapeDtypeStruct((TABLE_ROWS // 4, 128), jnp.float32),
    )(emb_t, wt, peb)


def _sc_gather_body(h2d, table, mu_out, idx2d, rows0, rows1, gsem, wsem):
    cid = lax.axis_index("c")
    sid = lax.axis_index("s")
    wid = sid * 2 + cid
    pltpu.sync_copy(h2d.at[pl.ds(wid * IDX_ROWS, IDX_ROWS)], idx2d)

    lane = lax.iota(jnp.int32, 16)

    def add_offs(j, carry):
        row = j // 8
        col = (j % 8) * 16
        offs = ((j % 16) * 16 + lane) * STRIDE
        r = idx2d[row, pl.ds(col, 16)] + offs
        w = jnp.bitwise_and(r, 8191)
        rp = (r - w) + jnp.left_shift(jnp.bitwise_and(w, 2047), 2) \
            + jnp.right_shift(w, 11)
        idx2d[row, pl.ds(col, 16)] = rp
        return carry

    lax.fori_loop(0, IDX_ROWS * 8, add_offs, 0)

    rows = [rows0, rows1]
    wb_handles = [None, None]
    for c in range(NCHUNK):
        buf = rows[c % 2]
        if wb_handles[c % 2] is not None:
            wb_handles[c % 2].wait()
        ghandles = []
        for k in range(GPC):
            ghandles.append(pltpu.async_copy(
                table.at[idx2d.at[c * GPC + k]],
                buf.at[pl.ds(k * 128, 128)],
                gsem))
        for h in ghandles:
            h.wait()
        a = (c // 2) % 4
        u0 = CHUNK * (c % 2)
        wb_handles[c % 2] = pltpu.async_copy(
            buf,
            mu_out.at[pl.ds(wid * 2048 + u0, CHUNK), pl.ds(32 * a, VAE_DIMS)],
            wsem)
    for h in wb_handles:
        if h is not None:
            h.wait()


def _sc_gather(h2d, table):
    mesh = plsc.VectorSubcoreMesh(core_axis_name="c", subcore_axis_name="s")
    f = functools.partial(
        pl.kernel,
        mesh=mesh,
        compiler_params=pltpu.CompilerParams(use_tc_tiling_on_sc=False),
        out_type=jax.ShapeDtypeStruct((ROWS_TOTAL // 4, 128), jnp.float32),
        scratch_types=[
            pltpu.VMEM((IDX_ROWS, 128), jnp.int32),
            pltpu.VMEM((CHUNK, VAE_DIMS), jnp.float32),
            pltpu.VMEM((CHUNK, VAE_DIMS), jnp.float32),
            pltpu.SemaphoreType.DMA,
            pltpu.SemaphoreType.DMA,
        ],
    )(_sc_gather_body)
    return f(h2d, table)


def _finish_body(m_ref, o_ref):
    m = m_ref[...]
    t = jnp.swapaxes(m, 0, 1)
    for a in range(4):
        for bb in range(8):
            o_ref[8 * a + bb] = t[32 * a:32 * (a + 1),
                                  256 * bb:256 * (bb + 1)]


def _finish(mu4):
    return pl.pallas_call(
        _finish_body,
        grid=(ROWS_TOTAL // A_COLS,),
        in_specs=[pl.BlockSpec((2048, 128), lambda i: (i, 0))],
        out_specs=pl.BlockSpec((32, OUT_DIMS, NUM_PARTS), lambda i: (i, 0, 0)),
        out_shape=jax.ShapeDtypeStruct((BATCH, OUT_DIMS, NUM_PARTS), jnp.float32),
    )(mu4)


def kernel(hashes, embedding, pe, W, b):
    B, P = hashes.shape
    h2d = hashes.reshape(-1, 128)
    emb_t = embedding.T
    peb = pe + b[None, :]
    p4 = _project(emb_t, W.T, peb)
    table = p4.reshape(TABLE_ROWS, VAE_DIMS)
    mu4 = _sc_gather(h2d, table)
    out_t = _finish(mu4)
    return jnp.swapaxes(out_t, 1, 2)
